# speedup vs baseline: 6.0868x; 6.0868x over previous
import jax
import jax.numpy as jnp
from jax.experimental import pallas as pl
from jax.experimental.pallas import tpu as pltpu

_B, _S, _P, _N = 32, 2048, 256, 256
_K, _L = 8, 4
_EPS = 1e-5


def _build_g_kernel(wb_ref, wenc_ref, g_ref):
    w = wb_ref[...]
    p = w
    for m in range(1, _K + _L):
        if m > 1:
            p = jnp.dot(w, p, precision=jax.lax.Precision.HIGHEST,
                        preferred_element_type=jnp.float32)
        for l in range(_L):
            k = _K - m + l
            if 0 <= k < _K:
                g_ref[l * _P:(l + 1) * _P, k * _P:(k + 1) * _P] = wenc_ref[l, k] * p


def _main_kernel(benc_ref, x_ref, g_ref, o_ref):
    x = x_ref[0]
    mean = jnp.mean(x, axis=0, keepdims=True)
    xc = x - mean
    var = jnp.mean(xc * xc, axis=0, keepdims=True)
    stdev = jnp.sqrt(var + _EPS)
    d = jnp.dot(g_ref[...], xc, preferred_element_type=jnp.float32)
    for l in range(_L):
        o_ref[0, l * _P:(l + 1) * _P, :] = (
            d[l * _P:(l + 1) * _P, :] + (benc_ref[l] * stdev + mean))


def kernel(x_enc, x_mark_enc, x_dec, x_mark_dec, W_base, W_enc, b_enc):
    g = pl.pallas_call(
        _build_g_kernel,
        out_shape=jax.ShapeDtypeStruct((_L * _P, _K * _P), jnp.float32),
        in_specs=[
            pl.BlockSpec(memory_space=pltpu.VMEM),
            pl.BlockSpec(memory_space=pltpu.SMEM),
        ],
        out_specs=pl.BlockSpec(memory_space=pltpu.VMEM),
        name="build_g",
    )(W_base, W_enc)

    out = pl.pallas_call(
        _main_kernel,
        grid=(_B,),
        out_shape=jax.ShapeDtypeStruct((_B, _L * _P, _N), jnp.float32),
        in_specs=[
            pl.BlockSpec(memory_space=pltpu.SMEM),
            pl.BlockSpec((1, _S, _N), lambda i: (i, 0, 0)),
            pl.BlockSpec((_L * _P, _K * _P), lambda i: (0, 0)),
        ],
        out_specs=pl.BlockSpec((1, _L * _P, _N), lambda i: (i, 0, 0)),
        compiler_params=pltpu.CompilerParams(
            dimension_semantics=("parallel",),
        ),
        name="seg_linear_main",
    )(b_enc, x_enc, g)
    return out


# speedup vs baseline: 6.6282x; 1.0889x over previous
import jax
import jax.numpy as jnp
from jax.experimental import pallas as pl
from jax.experimental.pallas import tpu as pltpu

_B, _S, _P, _N = 32, 2048, 256, 256
_K, _L = 8, 4
_EPS = 1e-5


def _main_kernel(benc_ref, wb_ref, wenc_ref, x_ref, o_ref, g_ref):
    @pl.when(pl.program_id(0) == 0)
    def _build_g():
        w = wb_ref[...]
        p = w
        for m in range(1, _K + _L):
            if m > 1:
                p = jnp.dot(w, p, precision=jax.lax.Precision.HIGHEST,
                            preferred_element_type=jnp.float32)
            for l in range(_L):
                k = _K - m + l
                if 0 <= k < _K:
                    g_ref[l * _P:(l + 1) * _P, k * _P:(k + 1) * _P] = (
                        wenc_ref[l, k] * p)

    x = x_ref[0]
    mean = jnp.mean(x, axis=0, keepdims=True)
    xc = x - mean
    var = jnp.mean(xc * xc, axis=0, keepdims=True)
    stdev = jnp.sqrt(var + _EPS)
    d = jnp.dot(g_ref[...], xc, preferred_element_type=jnp.float32)
    for l in range(_L):
        o_ref[0, l * _P:(l + 1) * _P, :] = (
            d[l * _P:(l + 1) * _P, :] + (benc_ref[l] * stdev + mean))


def kernel(x_enc, x_mark_enc, x_dec, x_mark_dec, W_base, W_enc, b_enc):
    out = pl.pallas_call(
        _main_kernel,
        grid=(_B,),
        out_shape=jax.ShapeDtypeStruct((_B, _L * _P, _N), jnp.float32),
        in_specs=[
            pl.BlockSpec(memory_space=pltpu.SMEM),
            pl.BlockSpec((_P, _P), lambda i: (0, 0)),
            pl.BlockSpec(memory_space=pltpu.SMEM),
            pl.BlockSpec((1, _S, _N), lambda i: (i, 0, 0)),
        ],
        out_specs=pl.BlockSpec((1, _L * _P, _N), lambda i: (i, 0, 0)),
        scratch_shapes=[pltpu.VMEM((_L * _P, _K * _P), jnp.float32)],
        compiler_params=pltpu.CompilerParams(
            dimension_semantics=("arbitrary",),
        ),
        name="seg_linear_main",
    )(b_enc, W_base, W_enc, x_enc)
    return out


# speedup vs baseline: 7.2560x; 1.0947x over previous
import jax
import jax.numpy as jnp
from jax.experimental import pallas as pl
from jax.experimental.pallas import tpu as pltpu

_B, _S, _P, _N = 32, 2048, 256, 256
_K, _L = 8, 4
_EPS = 1e-5


def _main_kernel(benc_ref, wb_ref, wenc_ref, x_ref, o_ref, g_ref, rs_ref):
    @pl.when(pl.program_id(0) == 0)
    def _build_g():
        w = wb_ref[...]
        p = w
        for m in range(1, _K + _L):
            if m > 1:
                p = jnp.dot(w, p, precision=jax.lax.Precision.HIGHEST,
                            preferred_element_type=jnp.float32)
            for l in range(_L):
                k = _K - m + l
                if 0 <= k < _K:
                    g_ref[l * _P:(l + 1) * _P, k * _P:(k + 1) * _P] = (
                        wenc_ref[l, k] * p)
        rs = jnp.sum(g_ref[...], axis=1, keepdims=True)
        rs_ref[...] = (1.0 - rs) * jnp.ones((1, _N), jnp.float32)

    x = x_ref[0]
    s1 = jnp.sum(x, axis=0, keepdims=True)
    s2 = jnp.sum(x * x, axis=0, keepdims=True)
    mean = s1 * (1.0 / _S)
    var = s2 * (1.0 / _S) - mean * mean
    stdev = jnp.sqrt(var + _EPS)
    d = jnp.dot(g_ref[...], x, preferred_element_type=jnp.float32)
    for l in range(_L):
        o_ref[0, l * _P:(l + 1) * _P, :] = (
            d[l * _P:(l + 1) * _P, :]
            + (rs_ref[l * _P:(l + 1) * _P, :] * mean + benc_ref[l] * stdev))


def kernel(x_enc, x_mark_enc, x_dec, x_mark_dec, W_base, W_enc, b_enc):
    out = pl.pallas_call(
        _main_kernel,
        grid=(_B,),
        out_shape=jax.ShapeDtypeStruct((_B, _L * _P, _N), jnp.float32),
        in_specs=[
            pl.BlockSpec(memory_space=pltpu.SMEM),
            pl.BlockSpec((_P, _P), lambda i: (0, 0)),
            pl.BlockSpec(memory_space=pltpu.SMEM),
            pl.BlockSpec((1, _S, _N), lambda i: (i, 0, 0)),
        ],
        out_specs=pl.BlockSpec((1, _L * _P, _N), lambda i: (i, 0, 0)),
        scratch_shapes=[
            pltpu.VMEM((_L * _P, _K * _P), jnp.float32),
            pltpu.VMEM((_L * _P, _N), jnp.float32),
        ],
        compiler_params=pltpu.CompilerParams(
            dimension_semantics=("arbitrary",),
        ),
        name="seg_linear_main",
    )(b_enc, W_base, W_enc, x_enc)
    return out


# speedup vs baseline: 8.6229x; 1.1884x over previous
import jax
import jax.numpy as jnp
from jax.experimental import pallas as pl
from jax.experimental.pallas import tpu as pltpu

_B, _S, _P, _N = 32, 2048, 256, 256
_K, _L = 8, 4
_NB = 2
_EPS = 1e-5


def _main_kernel(benc_ref, wb_ref, wenc_ref, x_ref, o_ref, g_ref, rs_ref):
    @pl.when(pl.program_id(0) == 0)
    def _build_g():
        w = wb_ref[...]
        p = w
        for m in range(1, _K + _L):
            if m > 1:
                p = jnp.dot(w, p, precision=jax.lax.Precision.HIGHEST,
                            preferred_element_type=jnp.float32)
            for l in range(_L):
                k = _K - m + l
                if 0 <= k < _K:
                    g_ref[l * _P:(l + 1) * _P, k * _P:(k + 1) * _P] = (
                        wenc_ref[l, k] * p)
        rs = jnp.sum(g_ref[...], axis=1, keepdims=True)
        rs_ref[...] = (1.0 - rs) * jnp.ones((1, _N), jnp.float32)

    for b in range(_NB):
        x = x_ref[b]
        s1 = jnp.sum(x, axis=0, keepdims=True)
        s2 = jnp.sum(x * x, axis=0, keepdims=True)
        mean = s1 * (1.0 / _S)
        var = s2 * (1.0 / _S) - mean * mean
        stdev = jnp.sqrt(var + _EPS)
        d = jnp.dot(g_ref[...], x, preferred_element_type=jnp.float32)
        for l in range(_L):
            o_ref[b, l * _P:(l + 1) * _P, :] = (
                d[l * _P:(l + 1) * _P, :]
                + (rs_ref[l * _P:(l + 1) * _P, :] * mean + benc_ref[l] * stdev))


def kernel(x_enc, x_mark_enc, x_dec, x_mark_dec, W_base, W_enc, b_enc):
    out = pl.pallas_call(
        _main_kernel,
        grid=(_B // _NB,),
        out_shape=jax.ShapeDtypeStruct((_B, _L * _P, _N), jnp.float32),
        in_specs=[
            pl.BlockSpec(memory_space=pltpu.SMEM),
            pl.BlockSpec((_P, _P), lambda i: (0, 0)),
            pl.BlockSpec(memory_space=pltpu.SMEM),
            pl.BlockSpec((_NB, _S, _N), lambda i: (i, 0, 0)),
        ],
        out_specs=pl.BlockSpec((_NB, _L * _P, _N), lambda i: (i, 0, 0)),
        scratch_shapes=[
            pltpu.VMEM((_L * _P, _K * _P), jnp.float32),
            pltpu.VMEM((_L * _P, _N), jnp.float32),
        ],
        compiler_params=pltpu.CompilerParams(
            dimension_semantics=("arbitrary",),
        ),
        name="seg_linear_main",
    )(b_enc, W_base, W_enc, x_enc)
    return out


# speedup vs baseline: 9.1518x; 1.0613x over previous
import jax
import jax.numpy as jnp
from jax.experimental import pallas as pl
from jax.experimental.pallas import tpu as pltpu

_B, _S, _P, _N = 32, 2048, 256, 256
_K, _L = 8, 4
_NB = 4
_EPS = 1e-5


def _main_kernel(benc_ref, wb_ref, wenc_ref, x_ref, o_ref, g_ref, rs_ref):
    @pl.when(pl.program_id(0) == 0)
    def _build_g():
        w = wb_ref[...]
        p = w
        for m in range(1, _K + _L):
            if m > 1:
                p = jnp.dot(w, p, precision=jax.lax.Precision.HIGHEST,
                            preferred_element_type=jnp.float32)
            for l in range(_L):
                k = _K - m + l
                if 0 <= k < _K:
                    g_ref[l * _P:(l + 1) * _P, k * _P:(k + 1) * _P] = (
                        wenc_ref[l, k] * p)
        rs = jnp.sum(g_ref[...], axis=1, keepdims=True)
        rs_ref[...] = (1.0 - rs) * jnp.ones((1, _N), jnp.float32)

    for b in range(_NB):
        x = x_ref[b]
        s1 = jnp.sum(x, axis=0, keepdims=True)
        s2 = jnp.sum(x * x, axis=0, keepdims=True)
        mean = s1 * (1.0 / _S)
        var = s2 * (1.0 / _S) - mean * mean
        stdev = jnp.sqrt(var + _EPS)
        d = jnp.dot(g_ref[...], x, preferred_element_type=jnp.float32)
        for l in range(_L):
            o_ref[b, l * _P:(l + 1) * _P, :] = (
                d[l * _P:(l + 1) * _P, :]
                + (rs_ref[l * _P:(l + 1) * _P, :] * mean + benc_ref[l] * stdev))


def kernel(x_enc, x_mark_enc, x_dec, x_mark_dec, W_base, W_enc, b_enc):
    out = pl.pallas_call(
        _main_kernel,
        grid=(_B // _NB,),
        out_shape=jax.ShapeDtypeStruct((_B, _L * _P, _N), jnp.float32),
        in_specs=[
            pl.BlockSpec(memory_space=pltpu.SMEM),
            pl.BlockSpec((_P, _P), lambda i: (0, 0)),
            pl.BlockSpec(memory_space=pltpu.SMEM),
            pl.BlockSpec((_NB, _S, _N), lambda i: (i, 0, 0)),
        ],
        out_specs=pl.BlockSpec((_NB, _L * _P, _N), lambda i: (i, 0, 0)),
        scratch_shapes=[
            pltpu.VMEM((_L * _P, _K * _P), jnp.float32),
            pltpu.VMEM((_L * _P, _N), jnp.float32),
        ],
        compiler_params=pltpu.CompilerParams(
            dimension_semantics=("arbitrary",),
        ),
        name="seg_linear_main",
    )(b_enc, W_base, W_enc, x_enc)
    return out
